# baseline (speedup 1.0000x reference)
import functools

import jax
import jax.numpy as jnp
from jax import lax
from jax.experimental import pallas as pl
from jax.experimental.pallas import tpu as pltpu
from jax.experimental.pallas import tpu_sc as plsc

N = 10000
R = 200
D = 768
E = 100000

NC = 2
NS = 16
NW = NC * NS
LG = D // 16
LG2 = LG // 2
DW = D // 2

RNG = 80
NUM_RANGES = N // RNG
RPW = -(-NUM_RANGES // NW)
NRP = 144
B = 32
MAXE = 2048
SCH = MAXE // B
EP = E + MAXE


def _vextract(vec_ref, idx):
    return vec_ref[pl.ds(idx, 16)][0]


def _sc_body(x_hbm, sig_hbm, src_hbm, typ_hbm, dst_hbm, st_hbm, nch_hbm,
             out_hbm, st_v, nch_v, src_v, typ_v, dst_v,
             rows0, sigr0, rows1, sigr1,
             acc_v, semi, semr0, sems0, semr1, sems1):
    c = lax.axis_index("c")
    s = lax.axis_index("s")
    wid = s * NC + c

    pltpu.sync_copy(st_hbm, st_v)
    pltpu.sync_copy(nch_hbm, nch_v)

    zero16 = jnp.zeros((16,), jnp.float32)
    rowbufs = ((rows0, sigr0, semr0, sems0), (rows1, sigr1, semr1, sems1))

    def _start_gather(b, off):
        rv, gv, sr, ss = rowbufs[b]
        sv = src_v.at[pl.ds(off, B)]
        tv = typ_v.at[pl.ds(off, B)]
        pltpu.make_async_copy(x_hbm.at[sv], rv, sr).start()
        pltpu.make_async_copy(sig_hbm.at[tv], gv, ss).start()

    def _wait_gather(b):
        rv, gv, sr, ss = rowbufs[b]
        sv = src_v.at[pl.ds(0, B)]
        tv = typ_v.at[pl.ds(0, B)]
        pltpu.make_async_copy(x_hbm.at[sv], rv, sr).wait()
        pltpu.make_async_copy(sig_hbm.at[tv], gv, ss).wait()

    def _compute(b, base, off):
        rv, gv, _, _ = rowbufs[b]

        for g in range(B // 16):
            loc = dst_v[pl.ds(off + 16 * g, 16)] - base
            okv = jnp.logical_and(loc >= 0, loc < RNG)
            locc = jnp.where(okv, loc, RNG)
            locs = [locc[l] for l in range(16)]

            @plsc.parallel_loop(0, LG2, 1, unroll=2)
            def _jloop(j):
                cs16 = pl.ds(16 * j, 16)
                for l in range(16):
                    row = 16 * g + l
                    av = plsc.bitcast(rv[row, cs16], jnp.bfloat16)
                    bv = plsc.bitcast(gv[row, cs16], jnp.bfloat16)
                    lo, hi = plsc.unpack(av * bv,
                                         format=plsc.PackFormat.INTERLEAVED)
                    plsc.addupdate(acc_v.at[locs[l], pl.ds(32 * j, 16)], lo)
                    plsc.addupdate(acc_v.at[locs[l], pl.ds(32 * j + 16, 16)],
                                   hi)

    def _range(ri, rcarry):
        r = wid + NW * ri

        @pl.when(r < NUM_RANGES)
        def _():
            base = r * RNG

            def _zero_row(i, carry):
                for j in range(LG):
                    acc_v[i, pl.ds(16 * j, 16)] = zero16
                return carry

            lax.fori_loop(0, RNG, _zero_row, 0)

            st = _vextract(st_v, r)
            nch = _vextract(nch_v, r)
            nsc = (nch + SCH - 1) // SCH

            def _super(si, scarry):
                e0 = pl.multiple_of(st + si * (SCH * B), 8)
                d1 = pltpu.make_async_copy(src_hbm.at[pl.ds(e0, MAXE)],
                                           src_v, semi)
                d2 = pltpu.make_async_copy(typ_hbm.at[pl.ds(e0, MAXE)],
                                           typ_v, semi)
                d3 = pltpu.make_async_copy(dst_hbm.at[pl.ds(e0, MAXE)],
                                           dst_v, semi)
                d1.start()
                d2.start()
                d3.start()
                d1.wait()
                d2.wait()
                d3.wait()
                cn = jnp.minimum(nch - si * SCH, SCH)

                _start_gather(0, 0)

                def _pair(k2, pcarry):
                    k = 2 * k2
                    _wait_gather(0)

                    @pl.when(k + 1 < cn)
                    def _():
                        _start_gather(1, (k + 1) * B)

                    _compute(0, base, k * B)

                    @pl.when(k + 1 < cn)
                    def _():
                        _wait_gather(1)

                        @pl.when(k + 2 < cn)
                        def _():
                            _start_gather(0, (k + 2) * B)

                        _compute(1, base, (k + 1) * B)

                    return pcarry

                lax.fori_loop(0, (cn + 1) // 2, _pair, 0)
                return scarry

            lax.fori_loop(0, nsc, _super, 0)

            pltpu.sync_copy(acc_v.at[pl.ds(0, RNG)],
                            out_hbm.at[pl.ds(base, RNG)])

        return rcarry

    lax.fori_loop(0, RPW, _range, 0)


def _sc_scatter(x, sig, src_s, typ_s, dst_s, starts, nchunks):
    mesh = plsc.VectorSubcoreMesh(core_axis_name="c", subcore_axis_name="s",
                                  num_cores=NC, num_subcores=NS)
    return pl.kernel(
        _sc_body,
        out_type=jax.ShapeDtypeStruct((N, D), jnp.float32),
        mesh=mesh,
        compiler_params=pltpu.CompilerParams(needs_layout_passes=False),
        scratch_types=[
            pltpu.VMEM((NRP,), jnp.int32),
            pltpu.VMEM((NRP,), jnp.int32),
            pltpu.VMEM((MAXE,), jnp.int32),
            pltpu.VMEM((MAXE,), jnp.int32),
            pltpu.VMEM((MAXE,), jnp.int32),
            pltpu.VMEM((B, DW), jnp.int32),
            pltpu.VMEM((B, DW), jnp.int32),
            pltpu.VMEM((B, DW), jnp.int32),
            pltpu.VMEM((B, DW), jnp.int32),
            pltpu.VMEM((RNG + 1, D), jnp.float32),
            pltpu.SemaphoreType.DMA,
            pltpu.SemaphoreType.DMA,
            pltpu.SemaphoreType.DMA,
            pltpu.SemaphoreType.DMA,
            pltpu.SemaphoreType.DMA,
        ],
    )(x, sig, src_s, typ_s, dst_s, starts, nchunks)


def _mm_body(x_ref, w_ref, o_ref):
    o_ref[...] = jnp.dot(x_ref[...], w_ref[...],
                         preferred_element_type=jnp.float32)


def _dense2_body(xw_ref, a_ref, wnt_ref, b_ref, o_ref, ob_ref):
    acc = xw_ref[...] + jnp.dot(a_ref[...], wnt_ref[...],
                                preferred_element_type=jnp.float32)
    o = jnp.maximum(acc + b_ref[...], 0.0)
    o_ref[...] = o
    ob_ref[...] = o.astype(jnp.bfloat16)


_BM = 400


def _mm(x, w):
    return pl.pallas_call(
        _mm_body,
        grid=(N // _BM,),
        in_specs=[
            pl.BlockSpec((_BM, D), lambda m: (m, 0)),
            pl.BlockSpec((D, D), lambda m: (0, 0)),
        ],
        out_specs=pl.BlockSpec((_BM, D), lambda m: (m, 0)),
        out_shape=jax.ShapeDtypeStruct((N, D), jnp.float32),
    )(x, w)


def _dense2(xw, agg, wnt, b2d):
    return pl.pallas_call(
        _dense2_body,
        grid=(N // _BM,),
        in_specs=[
            pl.BlockSpec((_BM, D), lambda m: (m, 0)),
            pl.BlockSpec((_BM, D), lambda m: (m, 0)),
            pl.BlockSpec((D, D), lambda m: (0, 0)),
            pl.BlockSpec((1, D), lambda m: (0, 0)),
        ],
        out_specs=(pl.BlockSpec((_BM, D), lambda m: (m, 0)),
                   pl.BlockSpec((_BM, D), lambda m: (m, 0))),
        out_shape=(jax.ShapeDtypeStruct((N, D), jnp.float32),
                   jax.ShapeDtypeStruct((N, D), jnp.bfloat16)),
    )(xw, agg, wnt, b2d)


def _rel_body(r_ref, wrt_ref, br_ref, sig_ref, r2_ref):
    rv = r_ref[...]
    sig_ref[...] = (1.0 / (1.0 + jnp.exp(-rv))).astype(jnp.bfloat16)
    r2_ref[...] = jnp.dot(rv, wrt_ref[...],
                          preferred_element_type=jnp.float32) + br_ref[...]


def _rel(r, wrt, br2d):
    return pl.pallas_call(
        _rel_body,
        out_shape=(jax.ShapeDtypeStruct((R, D), jnp.bfloat16),
                   jax.ShapeDtypeStruct((R, D), jnp.float32)),
    )(r, wrt, br2d)


import numpy as _np

_P = _np.arange(D)
_P = 32 * (_P // 32) + (_P % 2) * 16 + (_P % 32) // 2


def _pack_words(a_bf16):
    m = a_bf16.shape[0]
    return jax.lax.bitcast_convert_type(
        a_bf16.reshape(m, DW, 2), jnp.int32)


def _perm_bf16(a):
    return _pack_words(a[:, _P].astype(jnp.bfloat16))


def kernel(ent, rel, edge_index, edge_type, Ws0, bs0, Wn0, bn0, Wr0, br0,
           Ws1, bs1, Wn1, bn1, Wr1, br1):
    dst = edge_index[1]
    order = jnp.argsort(dst)
    src_s = edge_index[0][order]
    typ_s = edge_type[order]
    dst_s = dst[order]
    pad = EP - E
    src_p = jnp.concatenate([src_s, jnp.zeros((pad,), jnp.int32)])
    typ_p = jnp.concatenate([typ_s, jnp.zeros((pad,), jnp.int32)])
    dst_p = jnp.concatenate([dst_s, jnp.full((pad,), N, jnp.int32)])
    bounds = jnp.searchsorted(dst_s, jnp.arange(NUM_RANGES + 1,
                                                dtype=jnp.int32) * RNG)
    bounds = bounds.astype(jnp.int32)
    starts = bounds[:-1] & ~7
    nch = (bounds[1:] - starts + B - 1) // B
    starts = jnp.concatenate(
        [starts, jnp.zeros((NRP - NUM_RANGES,), jnp.int32)])
    nch = jnp.concatenate([nch, jnp.zeros((NRP - NUM_RANGES,), jnp.int32)])

    wst0 = Ws0.T[:, _P]
    wnt0 = Wn0.T[:, _P]
    b0 = (bs0 + bn0)[_P].reshape(1, D)
    wst1 = Ws1.T[_P]
    wnt1 = Wn1.T
    b1 = (bs1 + bn1).reshape(1, D)
    rel_p = rel[:, _P]
    wrt0 = Wr0.T[_P][:, _P]
    br0_2d = br0[_P].reshape(1, D)
    wrt1 = Wr1.T[_P]
    br1_2d = br1.reshape(1, D)

    sig0bf, r1p = _rel(rel_p, wrt0, br0_2d)
    xw0 = _mm(ent, wst0)
    agg0 = _sc_scatter(_perm_bf16(ent), _pack_words(sig0bf),
                       src_p, typ_p, dst_p, starts, nch)
    x1p, x1bf = _dense2(xw0, agg0, wnt0, b0)

    sig1bf, r2 = _rel(r1p, wrt1, br1_2d)
    xw1 = _mm(x1p, wst1)
    agg1 = _sc_scatter(_pack_words(x1bf), _pack_words(sig1bf),
                       src_p, typ_p, dst_p, starts, nch)
    x2, _ = _dense2(xw1, agg1, wnt1, b1)

    return (x2, r2)


# speedup vs baseline: 1.1468x; 1.1468x over previous
import functools

import jax
import jax.numpy as jnp
from jax import lax
from jax.experimental import pallas as pl
from jax.experimental.pallas import tpu as pltpu
from jax.experimental.pallas import tpu_sc as plsc

N = 10000
R = 200
D = 768
E = 100000

NC = 2
NS = 16
NW = NC * NS
LG = D // 16
LG2 = LG // 2
DW = D // 2

RNG = 80
NUM_RANGES = N // RNG
RPW = -(-NUM_RANGES // NW)
NRP = 144
B = 32
MAXE = 2048
SCH = MAXE // B
EP = E + MAXE


def _vextract(vec_ref, idx):
    return vec_ref[pl.ds(idx, 16)][0]


def _sc_body(x_hbm, sig_hbm, src_hbm, typ_hbm, dst_hbm, st_hbm, nch_hbm,
             out_hbm, st_v, nch_v, src_v, typ_v, dst_v,
             rows0, sigr0, rows1, sigr1,
             acc_v, semi, semr0, sems0, semr1, sems1):
    c = lax.axis_index("c")
    s = lax.axis_index("s")
    wid = s * NC + c

    pltpu.sync_copy(st_hbm, st_v)
    pltpu.sync_copy(nch_hbm, nch_v)

    zero16 = jnp.zeros((16,), jnp.float32)
    rowbufs = ((rows0, sigr0, semr0, sems0), (rows1, sigr1, semr1, sems1))

    def _start_gather(b, off):
        rv, gv, sr, ss = rowbufs[b]
        sv = src_v.at[pl.ds(off, B)]
        tv = typ_v.at[pl.ds(off, B)]
        pltpu.make_async_copy(x_hbm.at[sv], rv, sr).start()
        pltpu.make_async_copy(sig_hbm.at[tv], gv, ss).start()

    def _wait_gather(b):
        rv, gv, sr, ss = rowbufs[b]
        sv = src_v.at[pl.ds(0, B)]
        tv = typ_v.at[pl.ds(0, B)]
        pltpu.make_async_copy(x_hbm.at[sv], rv, sr).wait()
        pltpu.make_async_copy(sig_hbm.at[tv], gv, ss).wait()

    def _compute(b, base, off):
        rv, gv, _, _ = rowbufs[b]

        for g in range(B // 16):
            loc = dst_v[pl.ds(off + 16 * g, 16)] - base
            okv = jnp.logical_and(loc >= 0, loc < RNG)
            locc = jnp.where(okv, loc, RNG)
            locs = [locc[l] for l in range(16)]

            @plsc.parallel_loop(0, LG2, 1, unroll=2)
            def _jloop(j):
                cs16 = pl.ds(16 * j, 16)
                for l in range(16):
                    row = 16 * g + l
                    av = plsc.bitcast(rv[row, cs16], jnp.bfloat16)
                    bv = plsc.bitcast(gv[row, cs16], jnp.bfloat16)
                    lo, hi = plsc.unpack(av * bv,
                                         format=plsc.PackFormat.INTERLEAVED)
                    plsc.addupdate(acc_v.at[locs[l], pl.ds(32 * j, 16)], lo)
                    plsc.addupdate(acc_v.at[locs[l], pl.ds(32 * j + 16, 16)],
                                   hi)

    def _range(ri, rcarry):
        r = wid + NW * ri

        @pl.when(r < NUM_RANGES)
        def _():
            base = r * RNG

            def _zero_row(i, carry):
                for j in range(LG):
                    acc_v[i, pl.ds(16 * j, 16)] = zero16
                return carry

            lax.fori_loop(0, RNG, _zero_row, 0)

            st = _vextract(st_v, r)
            nch = _vextract(nch_v, r)
            nsc = (nch + SCH - 1) // SCH

            def _super(si, scarry):
                e0 = pl.multiple_of(st + si * (SCH * B), 8)
                d1 = pltpu.make_async_copy(src_hbm.at[pl.ds(e0, MAXE)],
                                           src_v, semi)
                d2 = pltpu.make_async_copy(typ_hbm.at[pl.ds(e0, MAXE)],
                                           typ_v, semi)
                d3 = pltpu.make_async_copy(dst_hbm.at[pl.ds(e0, MAXE)],
                                           dst_v, semi)
                d1.start()
                d2.start()
                d3.start()
                d1.wait()
                d2.wait()
                d3.wait()
                cn = jnp.minimum(nch - si * SCH, SCH)

                _start_gather(0, 0)

                def _pair(k2, pcarry):
                    k = 2 * k2
                    _wait_gather(0)

                    @pl.when(k + 1 < cn)
                    def _():
                        _start_gather(1, (k + 1) * B)

                    _compute(0, base, k * B)

                    @pl.when(k + 1 < cn)
                    def _():
                        _wait_gather(1)

                        @pl.when(k + 2 < cn)
                        def _():
                            _start_gather(0, (k + 2) * B)

                        _compute(1, base, (k + 1) * B)

                    return pcarry

                lax.fori_loop(0, (cn + 1) // 2, _pair, 0)
                return scarry

            lax.fori_loop(0, nsc, _super, 0)

            pltpu.sync_copy(acc_v.at[pl.ds(0, RNG)],
                            out_hbm.at[pl.ds(base, RNG)])

        return rcarry

    lax.fori_loop(0, RPW, _range, 0)


def _sc_scatter(x, sig, src_s, typ_s, dst_s, starts, nchunks):
    mesh = plsc.VectorSubcoreMesh(core_axis_name="c", subcore_axis_name="s",
                                  num_cores=NC, num_subcores=NS)
    return pl.kernel(
        _sc_body,
        out_type=jax.ShapeDtypeStruct((N, D), jnp.float32),
        mesh=mesh,
        compiler_params=pltpu.CompilerParams(needs_layout_passes=False),
        scratch_types=[
            pltpu.VMEM((NRP,), jnp.int32),
            pltpu.VMEM((NRP,), jnp.int32),
            pltpu.VMEM((MAXE,), jnp.int32),
            pltpu.VMEM((MAXE,), jnp.int32),
            pltpu.VMEM((MAXE,), jnp.int32),
            pltpu.VMEM((B, DW), jnp.int32),
            pltpu.VMEM((B, DW), jnp.int32),
            pltpu.VMEM((B, DW), jnp.int32),
            pltpu.VMEM((B, DW), jnp.int32),
            pltpu.VMEM((RNG + 1, D), jnp.float32),
            pltpu.SemaphoreType.DMA,
            pltpu.SemaphoreType.DMA,
            pltpu.SemaphoreType.DMA,
            pltpu.SemaphoreType.DMA,
            pltpu.SemaphoreType.DMA,
        ],
    )(x, sig, src_s, typ_s, dst_s, starts, nchunks)


def _mm_body(x_ref, w_ref, o_ref):
    o_ref[...] = jnp.dot(x_ref[...], w_ref[...],
                         preferred_element_type=jnp.float32)


def _dense2_body(xw_ref, a_ref, wnt_ref, b_ref, o_ref, ob_ref):
    acc = xw_ref[...] + jnp.dot(a_ref[...], wnt_ref[...],
                                preferred_element_type=jnp.float32)
    o = jnp.maximum(acc + b_ref[...], 0.0)
    o_ref[...] = o
    ob_ref[...] = o.astype(jnp.bfloat16)


_BM = 400


def _mm(x, w):
    return pl.pallas_call(
        _mm_body,
        grid=(N // _BM,),
        in_specs=[
            pl.BlockSpec((_BM, D), lambda m: (m, 0)),
            pl.BlockSpec((D, D), lambda m: (0, 0)),
        ],
        out_specs=pl.BlockSpec((_BM, D), lambda m: (m, 0)),
        out_shape=jax.ShapeDtypeStruct((N, D), jnp.float32),
    )(x, w)


def _dense2(xw, agg, wnt, b2d):
    return pl.pallas_call(
        _dense2_body,
        grid=(N // _BM,),
        in_specs=[
            pl.BlockSpec((_BM, D), lambda m: (m, 0)),
            pl.BlockSpec((_BM, D), lambda m: (m, 0)),
            pl.BlockSpec((D, D), lambda m: (0, 0)),
            pl.BlockSpec((1, D), lambda m: (0, 0)),
        ],
        out_specs=(pl.BlockSpec((_BM, D), lambda m: (m, 0)),
                   pl.BlockSpec((_BM, D), lambda m: (m, 0))),
        out_shape=(jax.ShapeDtypeStruct((N, D), jnp.float32),
                   jax.ShapeDtypeStruct((N, D), jnp.bfloat16)),
    )(xw, agg, wnt, b2d)


def _rel_body(r_ref, wrt_ref, br_ref, sig_ref, r2_ref):
    rv = r_ref[...]
    sig_ref[...] = (1.0 / (1.0 + jnp.exp(-rv))).astype(jnp.bfloat16)
    r2_ref[...] = jnp.dot(rv, wrt_ref[...],
                          preferred_element_type=jnp.float32) + br_ref[...]


def _rel(r, wrt, br2d):
    return pl.pallas_call(
        _rel_body,
        out_shape=(jax.ShapeDtypeStruct((R, D), jnp.bfloat16),
                   jax.ShapeDtypeStruct((R, D), jnp.float32)),
    )(r, wrt, br2d)


def _permc(a):
    m = a.shape[0]
    return a.reshape(m, LG2, 2, 16).swapaxes(2, 3).reshape(m, D)


def _permr(a):
    n = a.shape[1]
    return a.reshape(LG2, 2, 16, n).swapaxes(1, 2).reshape(D, n)


def _permv(v):
    return v.reshape(LG2, 2, 16).swapaxes(1, 2).reshape(D)


def _pack_words(a_bf16):
    m = a_bf16.shape[0]
    return jax.lax.bitcast_convert_type(
        a_bf16.reshape(m, DW, 2), jnp.int32)


def _perm_bf16(a):
    return _pack_words(_permc(a).astype(jnp.bfloat16))


def kernel(ent, rel, edge_index, edge_type, Ws0, bs0, Wn0, bn0, Wr0, br0,
           Ws1, bs1, Wn1, bn1, Wr1, br1):
    dst = edge_index[1]
    order = jnp.argsort(dst)
    src_s = edge_index[0][order]
    typ_s = edge_type[order]
    dst_s = dst[order]
    pad = EP - E
    src_p = jnp.concatenate([src_s, jnp.zeros((pad,), jnp.int32)])
    typ_p = jnp.concatenate([typ_s, jnp.zeros((pad,), jnp.int32)])
    dst_p = jnp.concatenate([dst_s, jnp.full((pad,), N, jnp.int32)])
    bounds = jnp.searchsorted(dst_s, jnp.arange(NUM_RANGES + 1,
                                                dtype=jnp.int32) * RNG)
    bounds = bounds.astype(jnp.int32)
    starts = bounds[:-1] & ~7
    nch = (bounds[1:] - starts + B - 1) // B
    starts = jnp.concatenate(
        [starts, jnp.zeros((NRP - NUM_RANGES,), jnp.int32)])
    nch = jnp.concatenate([nch, jnp.zeros((NRP - NUM_RANGES,), jnp.int32)])

    wst0 = _permc(Ws0.T)
    wnt0 = _permc(Wn0.T)
    b0 = _permv(bs0 + bn0).reshape(1, D)
    wst1 = _permr(Ws1.T)
    wnt1 = Wn1.T
    b1 = (bs1 + bn1).reshape(1, D)
    rel_p = _permc(rel)
    wrt0 = _permc(_permr(Wr0.T))
    br0_2d = _permv(br0).reshape(1, D)
    wrt1 = _permr(Wr1.T)
    br1_2d = br1.reshape(1, D)

    sig0bf, r1p = _rel(rel_p, wrt0, br0_2d)
    xw0 = _mm(ent, wst0)
    agg0 = _sc_scatter(_perm_bf16(ent), _pack_words(sig0bf),
                       src_p, typ_p, dst_p, starts, nch)
    x1p, x1bf = _dense2(xw0, agg0, wnt0, b0)

    sig1bf, r2 = _rel(r1p, wrt1, br1_2d)
    xw1 = _mm(x1p, wst1)
    agg1 = _sc_scatter(_pack_words(x1bf), _pack_words(sig1bf),
                       src_p, typ_p, dst_p, starts, nch)
    x2, _ = _dense2(xw1, agg1, wnt1, b1)

    return (x2, r2)


# speedup vs baseline: 1.4585x; 1.2718x over previous
import functools

import jax
import jax.numpy as jnp
from jax import lax
from jax.experimental import pallas as pl
from jax.experimental.pallas import tpu as pltpu
from jax.experimental.pallas import tpu_sc as plsc

N = 10000
R = 200
D = 768
E = 100000

NC = 2
NS = 16
NW = NC * NS
LG = D // 16
LG2 = LG // 2
DW = D // 2

RNG = 80
NUM_RANGES = N // RNG
RPW = -(-NUM_RANGES // NW)
NRP = 144
B = 32
MAXE = 2048
SCH = MAXE // B
EP = E + MAXE


def _vextract(vec_ref, idx):
    return vec_ref[pl.ds(idx, 16)][0]


def _sc_body(x_hbm, sig_hbm, src_hbm, typ_hbm, dst_hbm, st_hbm, nch_hbm,
             out_hbm, st_v, nch_v, src_v, typ_v, dst_v,
             rows0, sigr0, rows1, sigr1,
             acc_v, semi, semr0, sems0, semr1, sems1):
    c = lax.axis_index("c")
    s = lax.axis_index("s")
    wid = s * NC + c

    pltpu.sync_copy(st_hbm, st_v)
    pltpu.sync_copy(nch_hbm, nch_v)

    zero16 = jnp.zeros((16,), jnp.float32)
    rowbufs = ((rows0, sigr0, semr0, sems0), (rows1, sigr1, semr1, sems1))

    def _start_gather(b, off):
        rv, gv, sr, ss = rowbufs[b]
        sv = src_v.at[pl.ds(off, B)]
        tv = typ_v.at[pl.ds(off, B)]
        pltpu.make_async_copy(x_hbm.at[sv], rv, sr).start()
        pltpu.make_async_copy(sig_hbm.at[tv], gv, ss).start()

    def _wait_gather(b):
        rv, gv, sr, ss = rowbufs[b]
        sv = src_v.at[pl.ds(0, B)]
        tv = typ_v.at[pl.ds(0, B)]
        pltpu.make_async_copy(x_hbm.at[sv], rv, sr).wait()
        pltpu.make_async_copy(sig_hbm.at[tv], gv, ss).wait()

    def _compute(b, base, off):
        rv, gv, _, _ = rowbufs[b]

        for g in range(B // 16):
            loc = dst_v[pl.ds(off + 16 * g, 16)] - base
            okv = jnp.logical_and(loc >= 0, loc < RNG)
            locc = jnp.where(okv, loc, RNG)
            locs = [locc[l] for l in range(16)]

            @plsc.parallel_loop(0, LG2, 1, unroll=2)
            def _jloop(j):
                cs16 = pl.ds(16 * j, 16)
                for l in range(16):
                    row = 16 * g + l
                    av = plsc.bitcast(rv[row, cs16], jnp.bfloat16)
                    bv = plsc.bitcast(gv[row, cs16], jnp.bfloat16)
                    lo, hi = plsc.unpack(av * bv,
                                         format=plsc.PackFormat.INTERLEAVED)
                    plsc.addupdate(acc_v.at[locs[l], pl.ds(32 * j, 16)], lo)
                    plsc.addupdate(acc_v.at[locs[l], pl.ds(32 * j + 16, 16)],
                                   hi)

    def _range(ri, rcarry):
        r = wid + NW * ri

        @pl.when(r < NUM_RANGES)
        def _():
            base = r * RNG

            def _zero_row(i, carry):
                for j in range(LG):
                    acc_v[i, pl.ds(16 * j, 16)] = zero16
                return carry

            lax.fori_loop(0, RNG, _zero_row, 0)

            st = _vextract(st_v, r)
            nch = _vextract(nch_v, r)
            nsc = (nch + SCH - 1) // SCH

            def _super(si, scarry):
                e0 = pl.multiple_of(st + si * (SCH * B), 8)
                d1 = pltpu.make_async_copy(src_hbm.at[pl.ds(e0, MAXE)],
                                           src_v, semi)
                d2 = pltpu.make_async_copy(typ_hbm.at[pl.ds(e0, MAXE)],
                                           typ_v, semi)
                d3 = pltpu.make_async_copy(dst_hbm.at[pl.ds(e0, MAXE)],
                                           dst_v, semi)
                d1.start()
                d2.start()
                d3.start()
                d1.wait()
                d2.wait()
                d3.wait()
                cn = jnp.minimum(nch - si * SCH, SCH)

                _start_gather(0, 0)

                def _pair(k2, pcarry):
                    k = 2 * k2
                    _wait_gather(0)

                    @pl.when(k + 1 < cn)
                    def _():
                        _start_gather(1, (k + 1) * B)

                    _compute(0, base, k * B)

                    @pl.when(k + 1 < cn)
                    def _():
                        _wait_gather(1)

                        @pl.when(k + 2 < cn)
                        def _():
                            _start_gather(0, (k + 2) * B)

                        _compute(1, base, (k + 1) * B)

                    return pcarry

                lax.fori_loop(0, (cn + 1) // 2, _pair, 0)
                return scarry

            lax.fori_loop(0, nsc, _super, 0)

            pltpu.sync_copy(acc_v.at[pl.ds(0, RNG)],
                            out_hbm.at[pl.ds(base, RNG)])

        return rcarry

    lax.fori_loop(0, RPW, _range, 0)


def _sc_scatter(x, sig, src_s, typ_s, dst_s, starts, nchunks):
    mesh = plsc.VectorSubcoreMesh(core_axis_name="c", subcore_axis_name="s",
                                  num_cores=NC, num_subcores=NS)
    return pl.kernel(
        _sc_body,
        out_type=jax.ShapeDtypeStruct((N, D), jnp.float32),
        mesh=mesh,
        compiler_params=pltpu.CompilerParams(needs_layout_passes=False),
        scratch_types=[
            pltpu.VMEM((NRP,), jnp.int32),
            pltpu.VMEM((NRP,), jnp.int32),
            pltpu.VMEM((MAXE,), jnp.int32),
            pltpu.VMEM((MAXE,), jnp.int32),
            pltpu.VMEM((MAXE,), jnp.int32),
            pltpu.VMEM((B, DW), jnp.int32),
            pltpu.VMEM((B, DW), jnp.int32),
            pltpu.VMEM((B, DW), jnp.int32),
            pltpu.VMEM((B, DW), jnp.int32),
            pltpu.VMEM((RNG + 1, D), jnp.float32),
            pltpu.SemaphoreType.DMA,
            pltpu.SemaphoreType.DMA,
            pltpu.SemaphoreType.DMA,
            pltpu.SemaphoreType.DMA,
            pltpu.SemaphoreType.DMA,
        ],
    )(x, sig, src_s, typ_s, dst_s, starts, nchunks)


def _dense_body(x_ref, a_ref, wst_ref, wnt_ref, b_ref, o_ref):
    acc = jnp.dot(x_ref[...], wst_ref[...], preferred_element_type=jnp.float32)
    acc += jnp.dot(a_ref[...], wnt_ref[...], preferred_element_type=jnp.float32)
    o_ref[...] = jnp.maximum(acc + b_ref[...], 0.0)


_BM = 400


def _dense(x, agg, wst, wnt, b2d):
    return pl.pallas_call(
        _dense_body,
        grid=(N // _BM,),
        in_specs=[
            pl.BlockSpec((_BM, D), lambda m: (m, 0)),
            pl.BlockSpec((_BM, D), lambda m: (m, 0)),
            pl.BlockSpec((D, D), lambda m: (0, 0)),
            pl.BlockSpec((D, D), lambda m: (0, 0)),
            pl.BlockSpec((1, D), lambda m: (0, 0)),
        ],
        out_specs=pl.BlockSpec((_BM, D), lambda m: (m, 0)),
        out_shape=jax.ShapeDtypeStruct((N, D), jnp.float32),
    )(x, agg, wst, wnt, b2d)


def _rel_body(r_ref, wrt_ref, br_ref, sig_ref, r2_ref):
    rv = r_ref[...]
    sig_ref[...] = 1.0 / (1.0 + jnp.exp(-rv))
    r2_ref[...] = jnp.dot(rv, wrt_ref[...],
                          preferred_element_type=jnp.float32) + br_ref[...]


def _rel(r, wrt, br2d):
    return pl.pallas_call(
        _rel_body,
        out_shape=(jax.ShapeDtypeStruct((R, D), jnp.float32),
                   jax.ShapeDtypeStruct((R, D), jnp.float32)),
    )(r, wrt, br2d)


def _perm_bf16(a):
    m = a.shape[0]
    pb = (a.reshape(m, LG2, 2, 16).swapaxes(2, 3)
          .reshape(m, DW, 2).astype(jnp.bfloat16))
    return jax.lax.bitcast_convert_type(pb, jnp.int32)


def kernel(ent, rel, edge_index, edge_type, Ws0, bs0, Wn0, bn0, Wr0, br0,
           Ws1, bs1, Wn1, bn1, Wr1, br1):
    dst = edge_index[1]
    order = jnp.argsort(dst)
    src_s = edge_index[0][order]
    typ_s = edge_type[order]
    dst_s = dst[order]
    pad = EP - E
    src_p = jnp.concatenate([src_s, jnp.zeros((pad,), jnp.int32)])
    typ_p = jnp.concatenate([typ_s, jnp.zeros((pad,), jnp.int32)])
    dst_p = jnp.concatenate([dst_s, jnp.full((pad,), N, jnp.int32)])
    bounds = jnp.searchsorted(dst_s, jnp.arange(NUM_RANGES + 1,
                                                dtype=jnp.int32) * RNG)
    bounds = bounds.astype(jnp.int32)
    starts = bounds[:-1] & ~7
    nch = (bounds[1:] - starts + B - 1) // B
    starts = jnp.concatenate(
        [starts, jnp.zeros((NRP - NUM_RANGES,), jnp.int32)])
    nch = jnp.concatenate([nch, jnp.zeros((NRP - NUM_RANGES,), jnp.int32)])

    wst0, wnt0, wrt0 = Ws0.T, Wn0.T, Wr0.T
    wst1, wnt1, wrt1 = Ws1.T, Wn1.T, Wr1.T
    b0 = (bs0 + bn0).reshape(1, D)
    b1 = (bs1 + bn1).reshape(1, D)
    br0_2d = br0.reshape(1, D)
    br1_2d = br1.reshape(1, D)

    sig0, r1 = _rel(rel, wrt0, br0_2d)
    agg0 = _sc_scatter(_perm_bf16(ent), _perm_bf16(sig0),
                       src_p, typ_p, dst_p, starts, nch)
    x1 = _dense(ent, agg0, wst0, wnt0, b0)

    sig1, r2 = _rel(r1, wrt1, br1_2d)
    agg1 = _sc_scatter(_perm_bf16(x1), _perm_bf16(sig1),
                       src_p, typ_p, dst_p, starts, nch)
    x2 = _dense(x1, agg1, wst1, wnt1, b1)

    return (x2, r2)
